# baseline (device time: 53034 ns/iter reference)
import jax
import jax.numpy as jnp
from jax import lax
from jax.experimental import pallas as pl
from jax.experimental.pallas import tpu as pltpu

N_DEV = 4
NP = 2


def kernel(A, B, stage="full"):
    m, _ = A.shape
    _, n = B.shape
    chunk = m // N_DEV
    half = n // 2
    piece = chunk // NP

    def body(
        a_ref,
        b_ref,
        out_ref,
        b_bf,
        e1b, e2b, e3b, e4b, mlloc, mrloc, ownL, ownR,
        chainL, chainR, dirL, dirR, sumL, sumR, msendL, msendR,
        e1ss, e1rs, e2ss, e2rs, e3ss, e3rs, e4ss, e4rs,
        m1ss, m1rs, m2ss, m2rs,
        ag_ssem_r, ag_rsem_r, ag_ssem_l, ag_rsem_l,
    ):
        my = lax.axis_index("i")
        left = (my - 1) % N_DEV
        right = (my + 1) % N_DEV

        barrier_sem = pltpu.get_barrier_semaphore()
        for nbr in (left, right):
            pl.semaphore_signal(
                barrier_sem,
                inc=1,
                device_id=(nbr,),
                device_id_type=pl.DeviceIdType.MESH,
            )
        pl.semaphore_wait(barrier_sem, 2)

        def rdma(src, dst, ssem, rsem, target):
            return pltpu.make_async_remote_copy(
                src_ref=src, dst_ref=dst, send_sem=ssem, recv_sem=rsem,
                device_id=(target,), device_id_type=pl.DeviceIdType.MESH,
            )

        E1 = [rdma(e1b.at[p], chainL.at[p], e1ss.at[p], e1rs.at[p], right)
              for p in range(NP)]
        E2 = [rdma(e2b.at[p], chainR.at[p], e2ss.at[p], e2rs.at[p], left)
              for p in range(NP)]
        E3 = [rdma(e3b.at[p], dirL.at[p], e3ss.at[p], e3rs.at[p], left)
              for p in range(NP)]
        E4 = [rdma(e4b.at[p], dirR.at[p], e4ss.at[p], e4rs.at[p], right)
              for p in range(NP)]
        M1 = [rdma(msendL.at[p], sumL.at[p], m1ss.at[p], m1rs.at[p], right)
              for p in range(NP)]
        M2 = [rdma(msendR.at[p], sumR.at[p], m2ss.at[p], m2rs.at[p], left)
              for p in range(NP)]

        def out_sl(c, p, col0):
            return out_ref.at[
                pl.ds(c * chunk + p * piece, piece), pl.ds(col0, half)
            ]

        ag_send_r = [[rdma(out_sl((my - s) % N_DEV, p, 0),
                           out_sl((my - s) % N_DEV, p, 0),
                           ag_ssem_r.at[s, p], ag_rsem_r.at[s, p], right)
                      for p in range(NP)] for s in range(3)]
        ag_recv_r = [[rdma(out_sl((my - 1 - s) % N_DEV, p, 0),
                           out_sl((my - 1 - s) % N_DEV, p, 0),
                           ag_ssem_r.at[s, p], ag_rsem_r.at[s, p], right)
                      for p in range(NP)] for s in range(3)]
        ag_send_l = [[rdma(out_sl((my + s) % N_DEV, p, half),
                           out_sl((my + s) % N_DEV, p, half),
                           ag_ssem_l.at[s, p], ag_rsem_l.at[s, p], left)
                      for p in range(NP)] for s in range(3)]
        ag_recv_l = [[rdma(out_sl((my + 1 + s) % N_DEV, p, half),
                           out_sl((my + 1 + s) % N_DEV, p, half),
                           ag_ssem_l.at[s, p], ag_rsem_l.at[s, p], left)
                      for p in range(NP)] for s in range(3)]

        def dot_into(dst, c, p, col0):
            ap = a_ref[pl.ds(c * chunk + p * piece, piece), :].astype(
                jnp.bfloat16
            )
            dst[p, :, :] = jnp.dot(
                ap, b_bf[:, pl.ds(col0, half)],
                preferred_element_type=jnp.float32,
            ).astype(jnp.bfloat16)

        comm = stage != "mm"

        b_bf[:, :half] = b_ref[:, :half].astype(jnp.bfloat16)
        for p in range(NP):
            dot_into(e1b, (my + 2) % N_DEV, p, 0)
            if comm:
                E1[p].start()
        b_bf[:, half:] = b_ref[:, half:].astype(jnp.bfloat16)
        for p in range(NP):
            dot_into(e2b, (my + 2) % N_DEV, p, half)
            if comm:
                E2[p].start()
        for p in range(NP):
            dot_into(e4b, (my + 1) % N_DEV, p, half)
            if comm:
                E4[p].start()
        for p in range(NP):
            dot_into(e3b, (my - 1) % N_DEV, p, 0)
            if comm:
                E3[p].start()

        for p in range(NP):
            dot_into(mlloc, (my + 1) % N_DEV, p, 0)
            if comm:
                E1[p].wait_recv()
                msendL[p, :, :] = mlloc[p, :, :] + chainL[p, :, :]
                M1[p].start()
        for p in range(NP):
            dot_into(mrloc, (my - 1) % N_DEV, p, half)
            if comm:
                E2[p].wait_recv()
                msendR[p, :, :] = mrloc[p, :, :] + chainR[p, :, :]
                M2[p].start()

        for p in range(NP):
            dot_into(ownL, my, p, 0)
            if comm:
                M1[p].wait_recv()
                E3[p].wait_recv()
                zL = (
                    ownL[p, :, :].astype(jnp.float32)
                    + sumL[p, :, :].astype(jnp.float32)
                    + dirL[p, :, :].astype(jnp.float32)
                )
                out_ref[pl.ds(my * chunk + p * piece, piece), pl.ds(0, half)] = (
                    zL / (1.0 + jnp.exp(-zL))
                ).astype(jnp.bfloat16)
                if stage == "full":
                    ag_send_r[0][p].start()
        for p in range(NP):
            dot_into(ownR, my, p, half)
            if comm:
                M2[p].wait_recv()
                E4[p].wait_recv()
                zR = (
                    ownR[p, :, :].astype(jnp.float32)
                    + sumR[p, :, :].astype(jnp.float32)
                    + dirR[p, :, :].astype(jnp.float32)
                )
                out_ref[
                    pl.ds(my * chunk + p * piece, piece), pl.ds(half, half)
                ] = (zR / (1.0 + jnp.exp(-zR))).astype(jnp.bfloat16)
                if stage == "full":
                    ag_send_l[0][p].start()

        if stage == "mm":
            return

        if stage == "full":
            for s in range(3):
                for p in range(NP):
                    ag_recv_r[s][p].wait_recv()
                    if s < 2:
                        ag_send_r[s + 1][p].start()
                    ag_recv_l[s][p].wait_recv()
                    if s < 2:
                        ag_send_l[s + 1][p].start()

        for flow in (E1, E2, E3, E4, M1, M2):
            for op in flow:
                op.wait_send()
        if stage == "full":
            for grid in (ag_send_r, ag_send_l):
                for ops in grid:
                    for op in ops:
                        op.wait_send()

    return pl.pallas_call(
        body,
        out_shape=jax.ShapeDtypeStruct((m, n), jnp.bfloat16),
        in_specs=[
            pl.BlockSpec(memory_space=pltpu.VMEM),
            pl.BlockSpec(memory_space=pltpu.VMEM),
        ],
        out_specs=pl.BlockSpec(memory_space=pltpu.VMEM),
        scratch_shapes=[
            pltpu.VMEM(B.shape, jnp.bfloat16),
            pltpu.VMEM((NP, piece, half), jnp.bfloat16),
            pltpu.VMEM((NP, piece, half), jnp.bfloat16),
            pltpu.VMEM((NP, piece, half), jnp.bfloat16),
            pltpu.VMEM((NP, piece, half), jnp.bfloat16),
            pltpu.VMEM((NP, piece, half), jnp.bfloat16),
            pltpu.VMEM((NP, piece, half), jnp.bfloat16),
            pltpu.VMEM((NP, piece, half), jnp.bfloat16),
            pltpu.VMEM((NP, piece, half), jnp.bfloat16),
            pltpu.VMEM((NP, piece, half), jnp.bfloat16),
            pltpu.VMEM((NP, piece, half), jnp.bfloat16),
            pltpu.VMEM((NP, piece, half), jnp.bfloat16),
            pltpu.VMEM((NP, piece, half), jnp.bfloat16),
            pltpu.VMEM((NP, piece, half), jnp.bfloat16),
            pltpu.VMEM((NP, piece, half), jnp.bfloat16),
            pltpu.VMEM((NP, piece, half), jnp.bfloat16),
            pltpu.VMEM((NP, piece, half), jnp.bfloat16),
            pltpu.SemaphoreType.DMA((NP,)),
            pltpu.SemaphoreType.DMA((NP,)),
            pltpu.SemaphoreType.DMA((NP,)),
            pltpu.SemaphoreType.DMA((NP,)),
            pltpu.SemaphoreType.DMA((NP,)),
            pltpu.SemaphoreType.DMA((NP,)),
            pltpu.SemaphoreType.DMA((NP,)),
            pltpu.SemaphoreType.DMA((NP,)),
            pltpu.SemaphoreType.DMA((NP,)),
            pltpu.SemaphoreType.DMA((NP,)),
            pltpu.SemaphoreType.DMA((NP,)),
            pltpu.SemaphoreType.DMA((NP,)),
            pltpu.SemaphoreType.DMA((3, NP)),
            pltpu.SemaphoreType.DMA((3, NP)),
            pltpu.SemaphoreType.DMA((3, NP)),
            pltpu.SemaphoreType.DMA((3, NP)),
        ],
        compiler_params=pltpu.CompilerParams(collective_id=0),
    )(A, B)
